# baseline (speedup 1.0000x reference)
import functools

import jax
import jax.numpy as jnp
import numpy as np
from jax import lax
from jax.experimental import pallas as pl
from jax.experimental.pallas import tpu as pltpu
from jax.experimental.pallas import tpu_sc as plsc

B, L, D = 1, 2048, 2048
E, K, F = 8, 2, 1024
S = 2 * F
H = 4
HD = D // H
BR = 256
NSLOTS = 6144
NB = NSLOTS // BR
NW = 32


def _dot(a, b):
    return jnp.dot(a.astype(jnp.bfloat16), b.astype(jnp.bfloat16),
                   preferred_element_type=jnp.float32)


def _sigmoid(x):
    return 1.0 / (1.0 + jnp.exp(-x))


def _qkv_kernel(x_ref, nw_ref, w_ref, b_ref, o_ref):
    x = x_ref[...]
    v = jnp.mean(x * x, axis=-1, keepdims=True)
    xn = nw_ref[...] * (x * jax.lax.rsqrt(v + 1e-6))
    o_ref[...] = _dot(xn, w_ref[...].T) + b_ref[...]


def _qkv(x, nw, w, bias):
    RB, CB = 8, 6
    rb, cb = L // RB, (3 * D) // CB
    return pl.pallas_call(
        _qkv_kernel,
        grid=(CB, RB),
        in_specs=[
            pl.BlockSpec((rb, D), lambda c, r: (r, 0)),
            pl.BlockSpec((1, D), lambda c, r: (0, 0)),
            pl.BlockSpec((cb, D), lambda c, r: (c, 0)),
            pl.BlockSpec((1, cb), lambda c, r: (0, c)),
        ],
        out_specs=pl.BlockSpec((rb, cb), lambda c, r: (r, c)),
        out_shape=jax.ShapeDtypeStruct((L, 3 * D), jnp.float32),
    )(x, nw.reshape(1, D), w, bias.reshape(1, 3 * D))


def _attn_kernel(q_ref, k_ref, v_ref, o_ref):
    q = q_ref[...]
    k = k_ref[...]
    s = _dot(q, k.T) * (1.0 / np.sqrt(HD))
    s = s - jnp.max(s, axis=-1, keepdims=True)
    p = jnp.exp(s)
    den = jnp.sum(p, axis=-1, keepdims=True)
    o_ref[...] = _dot(p, v_ref[...]) / den


def _attn(qkv):
    QB = 2
    qb = L // QB
    return pl.pallas_call(
        _attn_kernel,
        grid=(H, QB),
        in_specs=[
            pl.BlockSpec((qb, HD), lambda h, q: (q, h)),
            pl.BlockSpec((L, HD), lambda h, q: (0, H + h)),
            pl.BlockSpec((L, HD), lambda h, q: (0, 2 * H + h)),
        ],
        out_specs=pl.BlockSpec((qb, HD), lambda h, q: (q, h)),
        out_shape=jax.ShapeDtypeStruct((L, D), jnp.float32),
    )(qkv, qkv, qkv)


def _outproj_kernel(ctx_ref, w_ref, b_ref, x_ref, o_ref):
    o_ref[...] = x_ref[...] + _dot(ctx_ref[...], w_ref[...].T) + b_ref[...]


def _outproj(ctx, w, bias, x):
    RB = 8
    rb = L // RB
    return pl.pallas_call(
        _outproj_kernel,
        grid=(RB,),
        in_specs=[
            pl.BlockSpec((rb, D), lambda r: (r, 0)),
            pl.BlockSpec((D, D), lambda r: (0, 0)),
            pl.BlockSpec((1, D), lambda r: (0, 0)),
            pl.BlockSpec((rb, D), lambda r: (r, 0)),
        ],
        out_specs=pl.BlockSpec((rb, D), lambda r: (r, 0)),
        out_shape=jax.ShapeDtypeStruct((L, D), jnp.float32),
    )(ctx, w, bias.reshape(1, D), x)


def _gate_kernel(hs_ref, gnw_ref, gw_ref, ogw_ref, ogb_ref,
                 base_ref, gl_ref, w1_ref, w2_ref, s1_ref, s2_ref, cnt_ref):
    hs = hs_ref[...]
    v = jnp.mean(hs * hs, axis=-1, keepdims=True)
    base = hs * jax.lax.rsqrt(v + 1e-6)
    base_ref[...] = base
    logits = _dot(base * gnw_ref[...], gw_ref[...].T)
    m = jnp.max(logits, axis=-1, keepdims=True)
    p = jnp.exp(logits - m)
    scores = p / jnp.sum(p, axis=-1, keepdims=True)
    iota = jax.lax.broadcasted_iota(jnp.int32, (L, E), 1)
    m1 = jnp.max(scores, axis=-1, keepdims=True)
    i1 = jnp.min(jnp.where(scores == m1, iota, E), axis=-1, keepdims=True)
    sc2 = jnp.where(iota == i1, -jnp.inf, scores)
    m2 = jnp.max(sc2, axis=-1, keepdims=True)
    i2 = jnp.min(jnp.where(sc2 == m2, iota, E), axis=-1, keepdims=True)
    wsum = m1 + m2 + 1e-20
    w1_ref[...] = jnp.broadcast_to(m1 / wsum, (L, 128))
    w2_ref[...] = jnp.broadcast_to(m2 / wsum, (L, 128))
    gl = jnp.sum(hs * ogw_ref[...], axis=-1, keepdims=True) + ogb_ref[0, 0]
    gl_ref[...] = jnp.broadcast_to(gl, (L, 128))
    one1 = iota == i1
    one2 = iota == i2
    mf = (one1 | one2).astype(jnp.bfloat16)
    ri = jax.lax.broadcasted_iota(jnp.int32, (L, L), 0)
    ci = jax.lax.broadcasted_iota(jnp.int32, (L, L), 1)
    tri = (ci < ri).astype(jnp.bfloat16)
    rank = _dot(tri, mf)
    counts = jnp.sum(mf.astype(jnp.float32), axis=0, keepdims=True)
    cnt_ref[...] = counts
    pc = jnp.ceil(counts * (1.0 / BR)) * BR
    ei = jax.lax.broadcasted_iota(jnp.int32, (E, E), 0)
    ej = jax.lax.broadcasted_iota(jnp.int32, (E, E), 1)
    tri_e = (ei < ej).astype(jnp.bfloat16)
    po = _dot(pc, tri_e)
    slot_all = jnp.broadcast_to(po, (L, E)) + rank
    s1_ref[...] = jnp.sum(jnp.where(one1, slot_all, 0.0), axis=-1,
                          keepdims=True).astype(jnp.int32)
    s2_ref[...] = jnp.sum(jnp.where(one2, slot_all, 0.0), axis=-1,
                          keepdims=True).astype(jnp.int32)


def _gate(hs, gnw, gw, ogw, ogb):
    return pl.pallas_call(
        _gate_kernel,
        grid=(1,),
        in_specs=[
            pl.BlockSpec((L, D), lambda i: (0, 0)),
            pl.BlockSpec((1, D), lambda i: (0, 0)),
            pl.BlockSpec((E, D), lambda i: (0, 0)),
            pl.BlockSpec((1, D), lambda i: (0, 0)),
            pl.BlockSpec((1, 1), lambda i: (0, 0)),
        ],
        out_specs=[
            pl.BlockSpec((L, D), lambda i: (0, 0)),
            pl.BlockSpec((L, 128), lambda i: (0, 0)),
            pl.BlockSpec((L, 128), lambda i: (0, 0)),
            pl.BlockSpec((L, 128), lambda i: (0, 0)),
            pl.BlockSpec((L, 1), lambda i: (0, 0)),
            pl.BlockSpec((L, 1), lambda i: (0, 0)),
            pl.BlockSpec((1, E), lambda i: (0, 0)),
        ],
        out_shape=[
            jax.ShapeDtypeStruct((L, D), jnp.float32),
            jax.ShapeDtypeStruct((L, 128), jnp.float32),
            jax.ShapeDtypeStruct((L, 128), jnp.float32),
            jax.ShapeDtypeStruct((L, 128), jnp.float32),
            jax.ShapeDtypeStruct((L, 1), jnp.int32),
            jax.ShapeDtypeStruct((L, 1), jnp.int32),
            jax.ShapeDtypeStruct((1, E), jnp.float32),
        ],
    )(hs, gnw.reshape(1, D), gw, ogw.reshape(1, D), ogb.reshape(1, 1))


def _dispatch(slot1, slot2, base):
    mesh = plsc.VectorSubcoreMesh(core_axis_name="c", subcore_axis_name="s")
    SPT = NSLOTS // NW
    CH = SPT // 16

    @functools.partial(
        pl.kernel, mesh=mesh,
        out_type=jax.ShapeDtypeStruct((NSLOTS, D), jnp.float32),
        compiler_params=pltpu.CompilerParams(needs_layout_passes=False),
        scratch_types=[
            pltpu.VMEM((L,), jnp.int32),
            pltpu.VMEM((L,), jnp.int32),
            pltpu.VMEM((NSLOTS,), jnp.int32),
            pltpu.VMEM_SHARED((NSLOTS,), jnp.int32),
            pltpu.VMEM((SPT,), jnp.int32),
            pltpu.VMEM((16, D), jnp.float32),
            pltpu.SemaphoreType.DMA,
        ])
    def disp(sl1_hbm, sl2_hbm, base_hbm, xs_hbm,
             sl1_v, sl2_v, tbs_v, tbs_sh, idx_v, rows_v, sem):
        c = lax.axis_index("c")
        s = lax.axis_index("s")

        @pl.when(s == 0)
        def _build():
            pltpu.sync_copy(sl1_hbm, sl1_v)
            pltpu.sync_copy(sl2_hbm, sl2_v)

            def zbody(i, carry):
                tbs_v[pl.ds(i * 16, 16)] = jnp.zeros((16,), jnp.int32)
                return carry

            lax.fori_loop(0, NSLOTS // 16, zbody, 0)
            lanes = lax.iota(jnp.int32, 16)

            def sbody(i, carry):
                toks = i * 16 + lanes
                a = sl1_v[pl.ds(i * 16, 16)]
                plsc.store_scatter(tbs_v, [a], toks)
                b2 = sl2_v[pl.ds(i * 16, 16)]
                plsc.store_scatter(tbs_v, [b2], toks)
                return carry

            lax.fori_loop(0, L // 16, sbody, 0)
            pltpu.sync_copy(tbs_v, tbs_sh)

        plsc.subcore_barrier()
        wid = c * 16 + s
        row0 = wid * SPT
        pltpu.sync_copy(tbs_sh.at[pl.ds(row0, SPT)], idx_v)

        def gbody(j, carry):
            pltpu.async_copy(
                base_hbm.at[idx_v.at[pl.ds(j * 16, 16)]], rows_v, sem).wait()
            pltpu.sync_copy(rows_v, xs_hbm.at[pl.ds(row0 + j * 16, 16)])
            return carry

        lax.fori_loop(0, CH, gbody, 0)

    return disp(slot1, slot2, base)


def _group_kernel(be_ref, bv_ref, xs_ref, enw_ref, wg_ref, wu_ref, wd_ref, o_ref):
    b = pl.program_id(0)

    @pl.when(bv_ref[b] == 1)
    def _():
        xn = (xs_ref[...] * enw_ref[0]).astype(jnp.bfloat16)
        g = _dot(xn, wg_ref[0].T)
        g = g * _sigmoid(g)
        u = _dot(xn, wu_ref[0].T)
        o_ref[...] = _dot((g * u).astype(jnp.bfloat16), wd_ref[0].T)


def _moe_group(be, bv, xs, enw, wg, wu, wd):
    grid_spec = pltpu.PrefetchScalarGridSpec(
        num_scalar_prefetch=2,
        grid=(NB,),
        in_specs=[
            pl.BlockSpec((BR, D), lambda b, be, bv: (b, 0)),
            pl.BlockSpec((1, D), lambda b, be, bv: (0, be[b])),
            pl.BlockSpec((1, F, D), lambda b, be, bv: (be[b], 0, 0)),
            pl.BlockSpec((1, F, D), lambda b, be, bv: (be[b], 0, 0)),
            pl.BlockSpec((1, D, F), lambda b, be, bv: (be[b], 0, 0)),
        ],
        out_specs=pl.BlockSpec((BR, D), lambda b, be, bv: (b, 0)),
    )
    return pl.pallas_call(
        _group_kernel,
        grid_spec=grid_spec,
        out_shape=jax.ShapeDtypeStruct((NSLOTS, D), jnp.float32),
    )(be, bv, xs, enw.reshape(1, E * D),
      wg.astype(jnp.bfloat16), wu.astype(jnp.bfloat16), wd.astype(jnp.bfloat16))


def _combine(eo, slot1r, slot2r):
    mesh = plsc.VectorSubcoreMesh(core_axis_name="c", subcore_axis_name="s")
    TPT = L // NW
    CH = TPT // 16

    @functools.partial(
        pl.kernel, mesh=mesh,
        out_type=[
            jax.ShapeDtypeStruct((L, D), jnp.float32),
            jax.ShapeDtypeStruct((L, D), jnp.float32),
        ],
        compiler_params=pltpu.CompilerParams(needs_layout_passes=False),
        scratch_types=[
            pltpu.VMEM((TPT,), jnp.int32),
            pltpu.VMEM((TPT,), jnp.int32),
            pltpu.VMEM((16, D), jnp.float32),
            pltpu.VMEM((16, D), jnp.float32),
            pltpu.SemaphoreType.DMA,
            pltpu.SemaphoreType.DMA,
        ])
    def comb(eo_hbm, sl1_hbm, sl2_hbm, ya_hbm, yb_hbm,
             i1_v, i2_v, r1_v, r2_v, sem1, sem2):
        c = lax.axis_index("c")
        s = lax.axis_index("s")
        wid = c * 16 + s
        t0 = wid * TPT
        pltpu.sync_copy(sl1_hbm.at[pl.ds(t0, TPT)], i1_v)
        pltpu.sync_copy(sl2_hbm.at[pl.ds(t0, TPT)], i2_v)

        def body(j, carry):
            cp1 = pltpu.async_copy(
                eo_hbm.at[i1_v.at[pl.ds(j * 16, 16)]], r1_v, sem1)
            cp2 = pltpu.async_copy(
                eo_hbm.at[i2_v.at[pl.ds(j * 16, 16)]], r2_v, sem2)
            cp1.wait()
            pltpu.sync_copy(r1_v, ya_hbm.at[pl.ds(t0 + j * 16, 16)])
            cp2.wait()
            pltpu.sync_copy(r2_v, yb_hbm.at[pl.ds(t0 + j * 16, 16)])
            return carry

        lax.fori_loop(0, CH, body, 0)

    return comb(eo, slot1r, slot2r)


def _shared1_kernel(x_ref, nw_ref, wg_ref, wu_ref, o_ref):
    x = x_ref[...]
    v = jnp.mean(x * x, axis=-1, keepdims=True)
    xn = (nw_ref[...] * (x * jax.lax.rsqrt(v + 1e-6))).astype(jnp.bfloat16)
    g = _dot(xn, wg_ref[...].T)
    o_ref[...] = g * _sigmoid(g) * _dot(xn, wu_ref[...].T)


def _shared1(x, nw, wg, wu):
    RB, CB = 8, 2
    rb, cb = L // RB, S // CB
    return pl.pallas_call(
        _shared1_kernel,
        grid=(CB, RB),
        in_specs=[
            pl.BlockSpec((rb, D), lambda c, r: (r, 0)),
            pl.BlockSpec((1, D), lambda c, r: (0, 0)),
            pl.BlockSpec((cb, D), lambda c, r: (c, 0)),
            pl.BlockSpec((cb, D), lambda c, r: (c, 0)),
        ],
        out_specs=pl.BlockSpec((rb, cb), lambda c, r: (r, c)),
        out_shape=jax.ShapeDtypeStruct((L, S), jnp.float32),
    )(x, nw.reshape(1, D), wg.astype(jnp.bfloat16), wu.astype(jnp.bfloat16))


def _final_kernel(s1_ref, wd_ref, ident_ref, ya_ref, yb_ref,
                  w1_ref, w2_ref, gl_ref, o_ref):
    shared = _dot(s1_ref[...], wd_ref[...].T)
    y = w1_ref[:, 0:1] * ya_ref[...] + w2_ref[:, 0:1] * yb_ref[...]
    gate = _sigmoid(gl_ref[:, 0:1])
    o_ref[...] = ident_ref[...] + gate * (y + shared)


def _final(s1, wd, ident, ya, yb, w1b, w2b, gl):
    RB = 8
    rb = L // RB
    return pl.pallas_call(
        _final_kernel,
        grid=(RB,),
        in_specs=[
            pl.BlockSpec((rb, S), lambda r: (r, 0)),
            pl.BlockSpec((D, S), lambda r: (0, 0)),
            pl.BlockSpec((rb, D), lambda r: (r, 0)),
            pl.BlockSpec((rb, D), lambda r: (r, 0)),
            pl.BlockSpec((rb, D), lambda r: (r, 0)),
            pl.BlockSpec((rb, 128), lambda r: (r, 0)),
            pl.BlockSpec((rb, 128), lambda r: (r, 0)),
            pl.BlockSpec((rb, 128), lambda r: (r, 0)),
        ],
        out_specs=pl.BlockSpec((rb, D), lambda r: (r, 0)),
        out_shape=jax.ShapeDtypeStruct((L, D), jnp.float32),
    )(s1, wd.astype(jnp.bfloat16), ident, ya, yb, w1b, w2b, gl)


def kernel(hidden_states, context_norm_w, attn_in_w, attn_in_b, attn_out_w,
           attn_out_b, gate_norm_w, gate_w, expert_norm_w, expert_wg,
           expert_wu, expert_wd, shared_norm_w, shared_wg, shared_wu,
           shared_wd, out_gate_w, out_gate_b):
    x = hidden_states.reshape(L, D)
    qkv = _qkv(x, context_norm_w, attn_in_w, attn_in_b)
    ctx = _attn(qkv)
    ident = _outproj(ctx, attn_out_w, attn_out_b, x)
    base, gl, w1b, w2b, slot1, slot2, counts = _gate(
        ident, gate_norm_w, gate_w, out_gate_w, out_gate_b)
    cnt = counts.reshape(E)
    pc = jnp.ceil(cnt / BR) * BR
    po = jnp.concatenate([jnp.zeros((1,), jnp.float32), jnp.cumsum(pc)[:-1]])
    bs = (jnp.arange(NB) * BR).astype(jnp.float32)
    ge = bs[None, :] >= po[:, None]
    be = jnp.clip(jnp.sum(ge.astype(jnp.int32), axis=0) - 1, 0, E - 1)
    bv = jnp.sum((ge & (bs[None, :] < (po + cnt)[:, None])).astype(jnp.int32),
                 axis=0)
    xs = _dispatch(slot1.reshape(L), slot2.reshape(L), base)
    s1 = _shared1(ident, shared_norm_w, shared_wg, shared_wu)
    eo = _moe_group(be, bv, xs, expert_norm_w, expert_wg, expert_wu, expert_wd)
    ya, yb = _combine(eo, slot1.reshape(L), slot2.reshape(L))
    out = _final(s1, shared_wd, ident, ya, yb, w1b, w2b, gl)
    return out.reshape(B, L, D)
